# baseline (speedup 1.0000x reference)
import jax
import jax.numpy as jnp
from jax import lax
from jax.experimental import pallas as pl

DEPTH = 1000
BATCH = 1024
GROUP = 26
ROWS = BATCH * GROUP
BLOCK_R = 512


def _onehot_body(idx_ref, out_ref):
    idx = idx_ref[...]
    iota = lax.broadcasted_iota(jnp.int32, (BLOCK_R, DEPTH), 1)
    out_ref[...] = (idx == iota).astype(jnp.float32)


def kernel(inputs):
    flat_idx = inputs.reshape(ROWS, 1)
    grid = (ROWS // BLOCK_R,)
    out = pl.pallas_call(
        _onehot_body,
        grid=grid,
        in_specs=[pl.BlockSpec((BLOCK_R, 1), lambda i: (i, 0))],
        out_specs=pl.BlockSpec((BLOCK_R, DEPTH), lambda i: (i, 0)),
        out_shape=jax.ShapeDtypeStruct((ROWS, DEPTH), jnp.float32),
    )(flat_idx)
    return out.reshape(BATCH, GROUP, DEPTH)
